# Initial kernel scaffold; baseline (speedup 1.0000x reference)
import jax
import jax.numpy as jnp
from jax.experimental import pallas as pl


def kernel(actions, state_indices, emb, W_ih, W_hh, b_ih, b_hh):
    raise NotImplementedError("write your pallas kernel here")


# speedup vs baseline: 3.3524x; 3.3524x over previous
import functools

import jax
import jax.numpy as jnp
from jax.experimental import pallas as pl
from jax.experimental.pallas import tpu as pltpu

B, T = 16, 512
ACTION_DIM, STATE_DIM, EMBED_DIM, H_DIM = 64, 1024, 128, 512
CT = 64
NT = T // CT

_PREC = jax.lax.Precision.HIGHEST


def _rnn_kernel(a_ref, idx_ref, emb_ref, w_a_ref, w_e_ref, w_hh_ref, bias_ref,
                out_ref, h_ref, z_ref):
    i = pl.program_id(0)

    @pl.when(i == 0)
    def _init():
        h_ref[...] = jnp.zeros_like(h_ref)

    idx = idx_ref[...]
    iota = jax.lax.broadcasted_iota(jnp.int32, (CT * B, STATE_DIM), 1)
    onehot = (idx == iota).astype(jnp.float32)
    s_emb = jax.lax.dot_general(
        onehot, emb_ref[...], (((1,), (0,)), ((), ())),
        preferred_element_type=jnp.float32, precision=_PREC)
    z = (jax.lax.dot_general(a_ref[...], w_a_ref[...], (((1,), (0,)), ((), ())),
                             preferred_element_type=jnp.float32, precision=_PREC)
         + jax.lax.dot_general(s_emb, w_e_ref[...], (((1,), (0,)), ((), ())),
                               preferred_element_type=jnp.float32, precision=_PREC)
         + bias_ref[...])
    z_ref[...] = z

    w_hh = w_hh_ref[...]

    def step(k, h):
        zk = z_ref[pl.ds(k * B, B), :]
        h_new = jnp.tanh(zk + jax.lax.dot_general(
            h, w_hh, (((1,), (0,)), ((), ())),
            preferred_element_type=jnp.float32, precision=_PREC))
        out_ref[pl.ds(k * B, B), :] = h_new
        return h_new

    h_ref[...] = jax.lax.fori_loop(0, CT, step, h_ref[...])


@jax.jit
def kernel(actions, state_indices, emb, W_ih, W_hh, b_ih, b_hh):
    a_tm = jnp.swapaxes(actions, 0, 1).reshape(T * B, ACTION_DIM)
    idx_tm = jnp.swapaxes(state_indices, 0, 1).reshape(T * B, 1).astype(jnp.int32)
    w_a = W_ih[:, :ACTION_DIM].T
    w_e = W_ih[:, ACTION_DIM:].T
    w_hh_t = W_hh.T
    bias = (b_ih + b_hh).reshape(1, H_DIM)

    out = pl.pallas_call(
        _rnn_kernel,
        grid=(NT,),
        in_specs=[
            pl.BlockSpec((CT * B, ACTION_DIM), lambda i: (i, 0)),
            pl.BlockSpec((CT * B, 1), lambda i: (i, 0)),
            pl.BlockSpec((STATE_DIM, EMBED_DIM), lambda i: (0, 0)),
            pl.BlockSpec((ACTION_DIM, H_DIM), lambda i: (0, 0)),
            pl.BlockSpec((EMBED_DIM, H_DIM), lambda i: (0, 0)),
            pl.BlockSpec((H_DIM, H_DIM), lambda i: (0, 0)),
            pl.BlockSpec((1, H_DIM), lambda i: (0, 0)),
        ],
        out_specs=pl.BlockSpec((CT * B, H_DIM), lambda i: (i, 0)),
        out_shape=jax.ShapeDtypeStruct((T * B, H_DIM), jnp.float32),
        scratch_shapes=[pltpu.VMEM((B, H_DIM), jnp.float32),
                        pltpu.VMEM((CT * B, H_DIM), jnp.float32)],
    )(a_tm, idx_tm, emb, w_a, w_e, w_hh_t, bias)

    return jnp.swapaxes(out.reshape(T, B, H_DIM), 0, 1)


# speedup vs baseline: 9.3953x; 2.8026x over previous
import functools

import jax
import jax.numpy as jnp
from jax.experimental import pallas as pl
from jax.experimental.pallas import tpu as pltpu

B, T = 16, 512
ACTION_DIM, STATE_DIM, EMBED_DIM, H_DIM = 64, 1024, 128, 512
CT = 64
NT = T // CT

_PREC = jax.lax.Precision.DEFAULT


def _rnn_kernel(a_ref, idx_ref, emb_ref, w_a_ref, w_e_ref, w_hh_ref, bias_ref,
                out_ref, h_ref, z_ref):
    i = pl.program_id(0)

    @pl.when(i == 0)
    def _init():
        h_ref[...] = jnp.zeros_like(h_ref)

    idx = idx_ref[...]
    iota = jax.lax.broadcasted_iota(jnp.int32, (CT * B, STATE_DIM), 1)
    onehot = (idx == iota).astype(jnp.float32)
    s_emb = jax.lax.dot_general(
        onehot, emb_ref[...], (((1,), (0,)), ((), ())),
        preferred_element_type=jnp.float32, precision=_PREC)
    z = (jax.lax.dot_general(a_ref[...], w_a_ref[...], (((1,), (0,)), ((), ())),
                             preferred_element_type=jnp.float32, precision=_PREC)
         + jax.lax.dot_general(s_emb, w_e_ref[...], (((1,), (0,)), ((), ())),
                               preferred_element_type=jnp.float32, precision=_PREC)
         + bias_ref[...])
    z_ref[...] = z

    w_hh = w_hh_ref[...]

    def step(k, h):
        zk = z_ref[pl.ds(k * B, B), :]
        h_new = jnp.tanh(zk + jax.lax.dot_general(
            h, w_hh, (((1,), (0,)), ((), ())),
            preferred_element_type=jnp.float32, precision=_PREC))
        out_ref[pl.ds(k * B, B), :] = h_new
        return h_new

    h_ref[...] = jax.lax.fori_loop(0, CT, step, h_ref[...])


@jax.jit
def kernel(actions, state_indices, emb, W_ih, W_hh, b_ih, b_hh):
    a_tm = jnp.swapaxes(actions, 0, 1).reshape(T * B, ACTION_DIM)
    idx_tm = jnp.swapaxes(state_indices, 0, 1).reshape(T * B, 1).astype(jnp.int32)
    w_a = W_ih[:, :ACTION_DIM].T
    w_e = W_ih[:, ACTION_DIM:].T
    w_hh_t = W_hh.T
    bias = (b_ih + b_hh).reshape(1, H_DIM)

    out = pl.pallas_call(
        _rnn_kernel,
        grid=(NT,),
        in_specs=[
            pl.BlockSpec((CT * B, ACTION_DIM), lambda i: (i, 0)),
            pl.BlockSpec((CT * B, 1), lambda i: (i, 0)),
            pl.BlockSpec((STATE_DIM, EMBED_DIM), lambda i: (0, 0)),
            pl.BlockSpec((ACTION_DIM, H_DIM), lambda i: (0, 0)),
            pl.BlockSpec((EMBED_DIM, H_DIM), lambda i: (0, 0)),
            pl.BlockSpec((H_DIM, H_DIM), lambda i: (0, 0)),
            pl.BlockSpec((1, H_DIM), lambda i: (0, 0)),
        ],
        out_specs=pl.BlockSpec((CT * B, H_DIM), lambda i: (i, 0)),
        out_shape=jax.ShapeDtypeStruct((T * B, H_DIM), jnp.float32),
        scratch_shapes=[pltpu.VMEM((B, H_DIM), jnp.float32),
                        pltpu.VMEM((CT * B, H_DIM), jnp.float32)],
    )(a_tm, idx_tm, emb, w_a, w_e, w_hh_t, bias)

    return jnp.swapaxes(out.reshape(T, B, H_DIM), 0, 1)


# speedup vs baseline: 11.6596x; 1.2410x over previous
import functools

import jax
import jax.numpy as jnp
from jax.experimental import pallas as pl
from jax.experimental.pallas import tpu as pltpu

B, T = 16, 512
ACTION_DIM, STATE_DIM, EMBED_DIM, H_DIM = 64, 1024, 128, 512
CT = 64
NT = T // CT

_PREC = jax.lax.Precision.DEFAULT


def _mm(a, b):
    return jax.lax.dot_general(a, b, (((1,), (0,)), ((), ())),
                               preferred_element_type=jnp.float32,
                               precision=_PREC)


def _rnn_kernel(a_ref, idx_ref, emb_ref, w_a_ref, w_e_ref, w_hh_ref, bias_ref,
                out_ref, h_ref, z_ref):
    i = pl.program_id(0)

    @pl.when(i == 0)
    def _init():
        h_ref[...] = jnp.zeros_like(h_ref)

    idx = idx_ref[...]
    iota = jax.lax.broadcasted_iota(jnp.int32, (CT * B, STATE_DIM), 1)
    onehot = (idx == iota).astype(jnp.float32)
    s_emb = _mm(onehot, emb_ref[...])
    z_ref[...] = (_mm(a_ref[...], w_a_ref[...])
                  + _mm(s_emb, w_e_ref[...])
                  + bias_ref[...])

    w_hh = w_hh_ref[...]
    h = h_ref[...]
    for k in range(CT):
        h = jnp.tanh(z_ref[k * B:(k + 1) * B, :] + _mm(h, w_hh))
        out_ref[:, k, :] = h
    h_ref[...] = h


@jax.jit
def kernel(actions, state_indices, emb, W_ih, W_hh, b_ih, b_hh):
    a_tm = jnp.swapaxes(actions, 0, 1).reshape(T * B, ACTION_DIM)
    idx_tm = jnp.swapaxes(state_indices, 0, 1).reshape(T * B, 1).astype(jnp.int32)
    w_a = W_ih[:, :ACTION_DIM].T
    w_e = W_ih[:, ACTION_DIM:].T
    w_hh_t = W_hh.T
    bias = (b_ih + b_hh).reshape(1, H_DIM)

    out = pl.pallas_call(
        _rnn_kernel,
        grid=(NT,),
        in_specs=[
            pl.BlockSpec((CT * B, ACTION_DIM), lambda i: (i, 0)),
            pl.BlockSpec((CT * B, 1), lambda i: (i, 0)),
            pl.BlockSpec((STATE_DIM, EMBED_DIM), lambda i: (0, 0)),
            pl.BlockSpec((ACTION_DIM, H_DIM), lambda i: (0, 0)),
            pl.BlockSpec((EMBED_DIM, H_DIM), lambda i: (0, 0)),
            pl.BlockSpec((H_DIM, H_DIM), lambda i: (0, 0)),
            pl.BlockSpec((1, H_DIM), lambda i: (0, 0)),
        ],
        out_specs=pl.BlockSpec((B, CT, H_DIM), lambda i: (0, i, 0)),
        out_shape=jax.ShapeDtypeStruct((B, T, H_DIM), jnp.float32),
        scratch_shapes=[pltpu.VMEM((B, H_DIM), jnp.float32),
                        pltpu.VMEM((CT * B, H_DIM), jnp.float32)],
    )(a_tm, idx_tm, emb, w_a, w_e, w_hh_t, bias)

    return out


# speedup vs baseline: 12.0966x; 1.0375x over previous
import functools

import jax
import jax.numpy as jnp
from jax.experimental import pallas as pl
from jax.experimental.pallas import tpu as pltpu

B, T = 16, 512
ACTION_DIM, STATE_DIM, EMBED_DIM, H_DIM = 64, 1024, 128, 512
CT = 64
NT = T // CT

_PREC = jax.lax.Precision.DEFAULT


def _mm(a, b):
    return jax.lax.dot_general(a, b, (((1,), (0,)), ((), ())),
                               preferred_element_type=jnp.float32,
                               precision=_PREC)


def _mmt(a, b):
    return jax.lax.dot_general(a, b, (((1,), (1,)), ((), ())),
                               preferred_element_type=jnp.float32,
                               precision=_PREC)


def _rnn_kernel(a_ref, idx_ref, emb_ref, w_ih_ref, w_hh_ref, b_ih_ref,
                b_hh_ref, out_ref, h_ref, z_ref):
    i = pl.program_id(0)

    @pl.when(i == 0)
    def _init():
        h_ref[...] = jnp.zeros_like(h_ref)

    @pl.when(i < NT)
    def _project():
        idx = idx_ref[...]
        iota = jax.lax.broadcasted_iota(jnp.int32, (CT * B, STATE_DIM), 1)
        onehot = (idx == iota).astype(jnp.float32)
        s_emb = _mm(onehot, emb_ref[...])
        z_ref[i % 2] = (_mmt(a_ref[...], w_ih_ref[:, :ACTION_DIM])
                        + _mmt(s_emb, w_ih_ref[:, ACTION_DIM:])
                        + b_ih_ref[...] + b_hh_ref[...])

    @pl.when(i > 0)
    def _recur():
        w_hh = w_hh_ref[...]
        zb = (i - 1) % 2
        h = h_ref[...]
        for k in range(CT):
            h = jnp.tanh(z_ref[zb, k * B:(k + 1) * B, :] + _mmt(h, w_hh))
            out_ref[:, k, :] = h
        h_ref[...] = h


@jax.jit
def kernel(actions, state_indices, emb, W_ih, W_hh, b_ih, b_hh):
    a_tm = jnp.swapaxes(actions, 0, 1).reshape(T * B, ACTION_DIM)
    idx_tm = jnp.swapaxes(state_indices, 0, 1).reshape(T * B, 1).astype(jnp.int32)

    last = NT - 1
    out = pl.pallas_call(
        _rnn_kernel,
        grid=(NT + 1,),
        in_specs=[
            pl.BlockSpec((CT * B, ACTION_DIM), lambda i: (jnp.minimum(i, last), 0)),
            pl.BlockSpec((CT * B, 1), lambda i: (jnp.minimum(i, last), 0)),
            pl.BlockSpec((STATE_DIM, EMBED_DIM), lambda i: (0, 0)),
            pl.BlockSpec((H_DIM, ACTION_DIM + EMBED_DIM), lambda i: (0, 0)),
            pl.BlockSpec((H_DIM, H_DIM), lambda i: (0, 0)),
            pl.BlockSpec((1, H_DIM), lambda i: (0, 0)),
            pl.BlockSpec((1, H_DIM), lambda i: (0, 0)),
        ],
        out_specs=pl.BlockSpec((B, CT, H_DIM),
                               lambda i: (0, jnp.maximum(i - 1, 0), 0)),
        out_shape=jax.ShapeDtypeStruct((B, T, H_DIM), jnp.float32),
        scratch_shapes=[pltpu.VMEM((B, H_DIM), jnp.float32),
                        pltpu.VMEM((2, CT * B, H_DIM), jnp.float32)],
    )(a_tm, idx_tm, emb, W_ih, W_hh,
      b_ih.reshape(1, H_DIM), b_hh.reshape(1, H_DIM))

    return out
